# Initial kernel scaffold; baseline (speedup 1.0000x reference)
import jax
import jax.numpy as jnp
from jax.experimental import pallas as pl


def kernel(input):
    raise NotImplementedError("write your pallas kernel here")


# speedup vs baseline: 43.0680x; 43.0680x over previous
import functools

import jax
import jax.numpy as jnp
from jax import lax
from jax.experimental import pallas as pl
from jax.experimental.pallas import tpu as pltpu
from jax.experimental.pallas import tpu_sc as plsc

_H = 256
_W = 256
_R2 = 1.5 * 1.5
_BACKGROUND = 1.0
_BIG = 1e9
_B = 32
_N = 8192
_L = 16
_GROUPS = _N // _L
_IMG = _H * _W
_IMG_PAD = _IMG + _L

_GATHER_DNUMS = lax.GatherDimensionNumbers(
    offset_dims=(), collapsed_slice_dims=(0,), start_index_map=(0,)
)


def _permute(x, idx):
    return lax.gather(
        x,
        idx[:, None],
        dimension_numbers=_GATHER_DNUMS,
        slice_sizes=(1,),
        mode=lax.GatherScatterMode.PROMISE_IN_BOUNDS,
    )


def _render_body(pts_hbm, out_hbm, pts_v, img_v):
    wid = lax.axis_index("s") * 2 + lax.axis_index("c")

    pltpu.sync_copy(pts_hbm.at[wid], pts_v)

    bg = jnp.full((_L,), _BACKGROUND, jnp.float32)

    def init_body(i, c):
        img_v[pl.ds(i * _L, _L)] = bg
        return c

    lax.fori_loop(0, _IMG_PAD // _L, init_body, 0)

    iota = lax.iota(jnp.int32, _L)
    stride3 = iota * 3
    nxt_down = jnp.maximum(iota - 1, 0)
    first_lane = iota == 0
    sentinel = iota + _IMG

    def body(g, c):
        base = g * (3 * _L)
        x = plsc.load_gather(pts_v, [stride3 + base])
        y = plsc.load_gather(pts_v, [stride3 + base + 1])
        z = plsc.load_gather(pts_v, [stride3 + base + 2])

        z, perm = plsc.sort_key_val(z, iota)
        x = _permute(x, perm)
        y = _permute(y, perm)

        xf = x * float(_W - 1)
        yf = y * float(_H - 1)
        ix = xf.astype(jnp.int32)
        iy = yf.astype(jnp.int32)
        fx = xf - ix.astype(jnp.float32)
        fy = yf - iy.astype(jnp.float32)
        bidx = iy * _W + ix

        xm = 1.0 + fx
        xp = 1.0 - fx
        ym = 1.0 + fy
        yp = 1.0 - fy
        ax = {-1: xm * xm, 0: fx * fx, 1: xp * xp}
        ay = {-1: ym * ym, 0: fy * fy, 1: yp * yp}
        okx = {-1: ix >= 1, 0: None, 1: ix <= _W - 2}
        oky = {-1: iy >= 1, 0: None, 1: iy <= _H - 2}

        for dy in (-1, 0, 1):
            for dx in (-1, 0, 1):
                d2 = ax[dx] + ay[dy]
                m = d2 <= _R2
                if okx[dx] is not None:
                    m = m & okx[dx]
                if oky[dy] is not None:
                    m = m & oky[dy]
                idx = bidx + (dy * _W + dx)
                idx_eff = jnp.where(m, idx, sentinel)
                z_eff = jnp.where(m, z, _BIG)
                key = idx_eff * _L + iota
                skey, sz = plsc.sort_key_val(key, z_eff)
                sidx = lax.shift_right_logical(skey, 4)
                prev = _permute(sidx, nxt_down)
                first = (prev != sidx) | first_lane
                cur = plsc.load_gather(img_v, [sidx])
                want = jnp.minimum(cur, sz)
                plsc.store_scatter(img_v, [sidx], want, mask=first)
        return c

    lax.fori_loop(0, _GROUPS, body, 0)

    pltpu.sync_copy(img_v.at[pl.ds(0, _IMG)], out_hbm.at[wid])


@jax.jit
def _render(pts):
    mesh = plsc.VectorSubcoreMesh(core_axis_name="c", subcore_axis_name="s")
    f = functools.partial(
        pl.kernel,
        out_type=jax.ShapeDtypeStruct((_B, _IMG), jnp.float32),
        mesh=mesh,
        compiler_params=pltpu.CompilerParams(needs_layout_passes=False),
        scratch_types=[
            pltpu.VMEM((3 * _N,), jnp.float32),
            pltpu.VMEM((_IMG_PAD,), jnp.float32),
        ],
    )(_render_body)
    return f(pts)


def kernel(input):
    pts = input.reshape(_B, 3 * _N)
    out = _render(pts)
    return out.reshape(_B, _H, _W)
